# Initial kernel scaffold; baseline (speedup 1.0000x reference)
import jax
import jax.numpy as jnp
from jax.experimental import pallas as pl


def kernel(x, Wg, bg, W1, b1, W2, b2):
    raise NotImplementedError("write your pallas kernel here")


# speedup vs baseline: 3.3182x; 3.3182x over previous
import functools
import math

import jax
import jax.numpy as jnp
from jax.experimental import pallas as pl
from jax.experimental.pallas import tpu as pltpu

B, S, D, H, E, K = 1, 2048, 768, 1536, 8, 2
T = B * S
M = 256
TB = T // M


def _top2_weights(scores):
    m_, e_ = scores.shape
    col = jax.lax.broadcasted_iota(jnp.int32, (m_, e_), 1)
    m1 = jnp.max(scores, axis=1, keepdims=True)
    idx1 = jnp.min(jnp.where(scores >= m1, col, e_), axis=1, keepdims=True)
    s2 = jnp.where(col == idx1, -jnp.inf, scores)
    m2 = jnp.max(s2, axis=1, keepdims=True)
    idx2 = jnp.min(jnp.where(s2 >= m2, col, e_), axis=1, keepdims=True)
    b = jnp.exp(m2 - m1)
    w1 = 1.0 / (1.0 + b)
    w2 = b / (1.0 + b)
    return jnp.where(col == idx1, w1, 0.0) + jnp.where(col == idx2, w2, 0.0)


def _gelu_exact(v):
    return 0.5 * v * (1.0 + jax.lax.erf(v * (1.0 / math.sqrt(2.0))))


def _moe_body(x_ref, wg_ref, bg_ref, w1_ref, b1_ref, w2_ref, b2_ref,
              out_ref, acc_ref, w_ref):
    e = pl.program_id(0)
    tb = pl.program_id(1)
    xb = x_ref[...]

    @pl.when(e == 0)
    def _():
        s = jnp.dot(xb.astype(jnp.bfloat16), wg_ref[...].astype(jnp.bfloat16),
                    preferred_element_type=jnp.float32) + bg_ref[...]
        w_ref[pl.ds(tb * M, M), :] = _top2_weights(s)

    w_all = w_ref[pl.ds(tb * M, M), :]
    col = jax.lax.broadcasted_iota(jnp.int32, (M, E), 1)
    wcol = jnp.sum(jnp.where(col == e, w_all, 0.0), axis=1, keepdims=True)

    xb16 = xb.astype(jnp.bfloat16)
    h = jnp.dot(xb16, w1_ref[0].astype(jnp.bfloat16),
                preferred_element_type=jnp.float32) + b1_ref[0]
    h = _gelu_exact(h)
    y = jnp.dot(h.astype(jnp.bfloat16), w2_ref[0].astype(jnp.bfloat16),
                preferred_element_type=jnp.float32) + b2_ref[0]
    contrib = wcol * y

    @pl.when(e == 0)
    def _():
        acc_ref[pl.ds(tb * M, M), :] = contrib

    @pl.when(e > 0)
    def _():
        acc_ref[pl.ds(tb * M, M), :] = acc_ref[pl.ds(tb * M, M), :] + contrib

    @pl.when(e == E - 1)
    def _():
        out_ref[...] = acc_ref[pl.ds(tb * M, M), :]


@jax.jit
def kernel(x, Wg, bg, W1, b1, W2, b2):
    x_flat = x.reshape(T, D)
    out = pl.pallas_call(
        _moe_body,
        grid=(E, TB),
        in_specs=[
            pl.BlockSpec((M, D), lambda e, tb: (tb, 0)),
            pl.BlockSpec((D, E), lambda e, tb: (0, 0)),
            pl.BlockSpec((1, E), lambda e, tb: (0, 0)),
            pl.BlockSpec((1, D, H), lambda e, tb: (e, 0, 0)),
            pl.BlockSpec((1, 1, H), lambda e, tb: (e, 0, 0)),
            pl.BlockSpec((1, H, D), lambda e, tb: (e, 0, 0)),
            pl.BlockSpec((1, 1, D), lambda e, tb: (e, 0, 0)),
        ],
        out_specs=pl.BlockSpec((M, D), lambda e, tb: (tb, 0)),
        out_shape=jax.ShapeDtypeStruct((T, D), jnp.float32),
        scratch_shapes=[
            pltpu.VMEM((T, D), jnp.float32),
            pltpu.VMEM((T, E), jnp.float32),
        ],
    )(x_flat, Wg, bg.reshape(1, E), W1, b1.reshape(E, 1, H), W2,
      b2.reshape(E, 1, D))
    return out.reshape(B, S, D)
